# baseline (speedup 1.0000x reference)
import jax
import jax.numpy as jnp
from jax.experimental import pallas as pl

_DIM = 256
_K = 8192
_NQ = 4
_T = 1152
_KC = 2048


def _argmin_body(r_ref, cb_ref, idx_ref):
    a = r_ref[...]
    a2 = jnp.sum(a * a, axis=1, keepdims=True)

    def chunk(c, carry):
        run_min, run_idx = carry
        b = cb_ref[pl.ds(c * _KC, _KC), :]
        b2 = jnp.sum(b * b, axis=1)
        dot = jax.lax.dot_general(
            a, b, (((1,), (1,)), ((), ())), preferred_element_type=jnp.float32
        )
        d = jnp.sqrt(jnp.maximum(a2 + b2[None, :] - 2.0 * dot, 0.0))
        lmin = jnp.min(d, axis=1)
        iota = jax.lax.broadcasted_iota(jnp.int32, d.shape, 1)
        lidx = jnp.min(jnp.where(d == lmin[:, None], iota, _K), axis=1) + c * _KC
        better = lmin < run_min
        return (
            jnp.where(better, lmin, run_min),
            jnp.where(better, lidx, run_idx),
        )

    init = (
        jnp.full((_T,), jnp.inf, dtype=jnp.float32),
        jnp.zeros((_T,), dtype=jnp.int32),
    )
    _, ridx = jax.lax.fori_loop(0, _K // _KC, chunk, init)
    idx_ref[0, 0, :] = ridx


def _argmin_round(r2d, cb):
    nt = r2d.shape[0] // _T
    idx = pl.pallas_call(
        _argmin_body,
        grid=(nt,),
        in_specs=[
            pl.BlockSpec((_T, _DIM), lambda i: (i, 0)),
            pl.BlockSpec((_K, _DIM), lambda i: (0, 0)),
        ],
        out_specs=pl.BlockSpec((1, 1, _T), lambda i: (i, 0, 0)),
        out_shape=jax.ShapeDtypeStruct((nt, 1, _T), jnp.int32),
    )(r2d, cb)
    return idx.reshape(-1)


def kernel(x, codebooks):
    shp = x.shape
    r = x.reshape(-1, _DIM)
    qout = jnp.zeros_like(r)
    idxs = []
    losses = []
    for q in range(_NQ):
        cb = codebooks[q]
        idx = _argmin_round(r, cb)
        quant = jnp.take(cb, idx, axis=0)
        q_st = r + (quant - r)
        losses.append(jnp.mean((r - quant) ** 2))
        r = r - q_st
        qout = qout + q_st
        idxs.append(idx.reshape(shp[:-1]))
    return (
        qout.reshape(shp),
        jnp.stack(idxs, axis=-1),
        jnp.stack(losses, axis=-1),
    )


# speedup vs baseline: 1.4544x; 1.4544x over previous
import jax
import jax.numpy as jnp
from jax import lax
from jax.experimental import pallas as pl

_DIM = 256
_K = 8192
_NQ = 4
_T = 1152
_KC = 2048


def _b2_body(cb_ref, b2_ref):
    c = cb_ref[0]
    b2_ref[0, 0, :] = jnp.sum(c * c, axis=-1)


def _codebook_sqnorms(codebooks):
    return pl.pallas_call(
        _b2_body,
        grid=(_NQ,),
        in_specs=[pl.BlockSpec((1, _K, _DIM), lambda q: (q, 0, 0))],
        out_specs=pl.BlockSpec((1, 1, _K), lambda q: (q, 0, 0)),
        out_shape=jax.ShapeDtypeStruct((_NQ, 1, _K), jnp.float32),
    )(codebooks)


def _scan_codebook(r, cb_ref, b2_ref):
    r2 = r + r
    a2 = jnp.sum(r * r, axis=1)[:, None]

    def chunk(c, carry):
        best_d, best_i = carry
        b = cb_ref[pl.ds(c * _KC, _KC), :]
        dot2 = lax.dot_general(
            r2, b, (((1,), (1,)), ((), ())), preferred_element_type=jnp.float32
        )
        s = (a2 + b2_ref[0, pl.ds(c * _KC, _KC)][None, :]) - dot2
        m = jnp.min(s, axis=1)
        iota = lax.broadcasted_iota(jnp.int32, s.shape, 1)
        li = jnp.min(jnp.where(s == m[:, None], iota, _KC), axis=1) + c * _KC
        better = m < best_d
        return (
            jnp.where(better, m, best_d),
            jnp.where(better, li, best_i),
        )

    init = (
        jnp.full((_T,), jnp.inf, dtype=jnp.float32),
        jnp.zeros((_T,), dtype=jnp.int32),
    )
    _, best_i = lax.fori_loop(0, _K // _KC, chunk, init)
    return best_i


def _round0_body(r_ref, cb_ref, b2_ref, idx_ref):
    idx_ref[0, 0, :] = _scan_codebook(r_ref[...], cb_ref, b2_ref)


def _round_body(r_ref, q_ref, cb_ref, b2_ref, idx_ref, rnew_ref, loss_ref):
    r = r_ref[...]
    quant = q_ref[...]
    loss_ref[...] = jnp.sum((r - quant) ** 2).reshape(1, 1, 1)
    q_st = r + (quant - r)
    rn = r - q_st
    rnew_ref[...] = rn
    idx_ref[0, 0, :] = _scan_codebook(rn, cb_ref, b2_ref)


def _final_body(r_ref, q_ref, rfin_ref, loss_ref):
    r = r_ref[...]
    quant = q_ref[...]
    loss_ref[...] = jnp.sum((r - quant) ** 2).reshape(1, 1, 1)
    q_st = r + (quant - r)
    rfin_ref[...] = r - q_st


def _round0(r, cb, b2):
    nt = r.shape[0] // _T
    return pl.pallas_call(
        _round0_body,
        grid=(nt,),
        in_specs=[
            pl.BlockSpec((_T, _DIM), lambda i: (i, 0)),
            pl.BlockSpec((_K, _DIM), lambda i: (0, 0)),
            pl.BlockSpec((1, _K), lambda i: (0, 0)),
        ],
        out_specs=pl.BlockSpec((1, 1, _T), lambda i: (i, 0, 0)),
        out_shape=jax.ShapeDtypeStruct((nt, 1, _T), jnp.int32),
    )(r, cb, b2)


def _round_upd(r, quant, cb, b2):
    nt = r.shape[0] // _T
    return pl.pallas_call(
        _round_body,
        grid=(nt,),
        in_specs=[
            pl.BlockSpec((_T, _DIM), lambda i: (i, 0)),
            pl.BlockSpec((_T, _DIM), lambda i: (i, 0)),
            pl.BlockSpec((_K, _DIM), lambda i: (0, 0)),
            pl.BlockSpec((1, _K), lambda i: (0, 0)),
        ],
        out_specs=[
            pl.BlockSpec((1, 1, _T), lambda i: (i, 0, 0)),
            pl.BlockSpec((_T, _DIM), lambda i: (i, 0)),
            pl.BlockSpec((1, 1, 1), lambda i: (i, 0, 0)),
        ],
        out_shape=[
            jax.ShapeDtypeStruct((nt, 1, _T), jnp.int32),
            jax.ShapeDtypeStruct((r.shape[0], _DIM), jnp.float32),
            jax.ShapeDtypeStruct((nt, 1, 1), jnp.float32),
        ],
    )(r, quant, cb, b2)


def _final_upd(r, quant):
    nt = r.shape[0] // _T
    return pl.pallas_call(
        _final_body,
        grid=(nt,),
        in_specs=[
            pl.BlockSpec((_T, _DIM), lambda i: (i, 0)),
            pl.BlockSpec((_T, _DIM), lambda i: (i, 0)),
        ],
        out_specs=[
            pl.BlockSpec((_T, _DIM), lambda i: (i, 0)),
            pl.BlockSpec((1, 1, 1), lambda i: (i, 0, 0)),
        ],
        out_shape=[
            jax.ShapeDtypeStruct((r.shape[0], _DIM), jnp.float32),
            jax.ShapeDtypeStruct((nt, 1, 1), jnp.float32),
        ],
    )(r, quant)


def kernel(x, codebooks):
    shp = x.shape
    n = x.size // _DIM
    r0 = x.reshape(n, _DIM)
    b2 = _codebook_sqnorms(codebooks)

    idxs = [_round0(r0, codebooks[0], b2[0]).reshape(n)]
    loss_parts = []
    r = r0
    for q in range(1, _NQ):
        quant = jnp.take(codebooks[q - 1], idxs[-1], axis=0)
        idx, r, lp = _round_upd(r, quant, codebooks[q], b2[q])
        idxs.append(idx.reshape(n))
        loss_parts.append(lp)
    quant = jnp.take(codebooks[_NQ - 1], idxs[-1], axis=0)
    rfin, lp = _final_upd(r, quant)
    loss_parts.append(lp)

    qout = (r0 - rfin).reshape(shp)
    indices = jnp.stack([i.reshape(shp[:-1]) for i in idxs], axis=-1)
    denom = float(n * _DIM)
    losses = jnp.stack([jnp.sum(p) / denom for p in loss_parts], axis=-1)
    return qout, indices, losses


# speedup vs baseline: 1.5616x; 1.0737x over previous
import jax
import jax.numpy as jnp
from jax import lax
from jax.experimental import pallas as pl

_DIM = 256
_K = 8192
_NQ = 4
_T = 1152
_KC = 2048


def _b2_body(cb_ref, b2_ref):
    c = cb_ref[0]
    b2_ref[0, 0, :] = jnp.sum(c * c, axis=-1)


def _codebook_sqnorms(codebooks):
    return pl.pallas_call(
        _b2_body,
        grid=(_NQ,),
        in_specs=[pl.BlockSpec((1, _K, _DIM), lambda q: (q, 0, 0))],
        out_specs=pl.BlockSpec((1, 1, _K), lambda q: (q, 0, 0)),
        out_shape=jax.ShapeDtypeStruct((_NQ, 1, _K), jnp.float32),
    )(codebooks)


def _scan_codebook(r, cb_ref, b2_ref):
    r2 = r + r
    a2 = jnp.sum(r * r, axis=1)[:, None]

    def chunk(c, carry):
        best_d, best_i = carry
        b = cb_ref[pl.ds(c * _KC, _KC), :]
        dot2 = lax.dot_general(
            r2, b, (((1,), (1,)), ((), ())), preferred_element_type=jnp.float32
        )
        s = (a2 + b2_ref[0, pl.ds(c * _KC, _KC)][None, :]) - dot2
        m = jnp.min(s, axis=1)
        iota = lax.broadcasted_iota(jnp.int32, s.shape, 1)
        li = jnp.min(jnp.where(s == m[:, None], iota, _KC), axis=1) + c * _KC
        better = m < best_d
        return (
            jnp.where(better, m, best_d),
            jnp.where(better, li, best_i),
        )

    carry = (
        jnp.full((_T,), jnp.inf, dtype=jnp.float32),
        jnp.zeros((_T,), dtype=jnp.int32),
    )
    for c in range(_K // _KC):
        carry = chunk(c, carry)
    return carry[1]


def _round0_body(r_ref, cb_ref, b2_ref, idx_ref):
    idx_ref[0, 0, :] = _scan_codebook(r_ref[...], cb_ref, b2_ref)


def _round_body(r_ref, q_ref, cb_ref, b2_ref, idx_ref, rnew_ref, loss_ref):
    r = r_ref[...]
    quant = q_ref[...]
    loss_ref[...] = jnp.sum((r - quant) ** 2).reshape(1, 1, 1)
    q_st = r + (quant - r)
    rn = r - q_st
    rnew_ref[...] = rn
    idx_ref[0, 0, :] = _scan_codebook(rn, cb_ref, b2_ref)


def _final_body(r_ref, q_ref, rfin_ref, loss_ref):
    r = r_ref[...]
    quant = q_ref[...]
    loss_ref[...] = jnp.sum((r - quant) ** 2).reshape(1, 1, 1)
    q_st = r + (quant - r)
    rfin_ref[...] = r - q_st


def _round0(r, cb, b2):
    nt = r.shape[0] // _T
    return pl.pallas_call(
        _round0_body,
        grid=(nt,),
        in_specs=[
            pl.BlockSpec((_T, _DIM), lambda i: (i, 0)),
            pl.BlockSpec((_K, _DIM), lambda i: (0, 0)),
            pl.BlockSpec((1, _K), lambda i: (0, 0)),
        ],
        out_specs=pl.BlockSpec((1, 1, _T), lambda i: (i, 0, 0)),
        out_shape=jax.ShapeDtypeStruct((nt, 1, _T), jnp.int32),
    )(r, cb, b2)


def _round_upd(r, quant, cb, b2):
    nt = r.shape[0] // _T
    return pl.pallas_call(
        _round_body,
        grid=(nt,),
        in_specs=[
            pl.BlockSpec((_T, _DIM), lambda i: (i, 0)),
            pl.BlockSpec((_T, _DIM), lambda i: (i, 0)),
            pl.BlockSpec((_K, _DIM), lambda i: (0, 0)),
            pl.BlockSpec((1, _K), lambda i: (0, 0)),
        ],
        out_specs=[
            pl.BlockSpec((1, 1, _T), lambda i: (i, 0, 0)),
            pl.BlockSpec((_T, _DIM), lambda i: (i, 0)),
            pl.BlockSpec((1, 1, 1), lambda i: (i, 0, 0)),
        ],
        out_shape=[
            jax.ShapeDtypeStruct((nt, 1, _T), jnp.int32),
            jax.ShapeDtypeStruct((r.shape[0], _DIM), jnp.float32),
            jax.ShapeDtypeStruct((nt, 1, 1), jnp.float32),
        ],
    )(r, quant, cb, b2)


def _final_upd(r, quant):
    nt = r.shape[0] // _T
    return pl.pallas_call(
        _final_body,
        grid=(nt,),
        in_specs=[
            pl.BlockSpec((_T, _DIM), lambda i: (i, 0)),
            pl.BlockSpec((_T, _DIM), lambda i: (i, 0)),
        ],
        out_specs=[
            pl.BlockSpec((_T, _DIM), lambda i: (i, 0)),
            pl.BlockSpec((1, 1, 1), lambda i: (i, 0, 0)),
        ],
        out_shape=[
            jax.ShapeDtypeStruct((r.shape[0], _DIM), jnp.float32),
            jax.ShapeDtypeStruct((nt, 1, 1), jnp.float32),
        ],
    )(r, quant)


def kernel(x, codebooks):
    shp = x.shape
    n = x.size // _DIM
    r0 = x.reshape(n, _DIM)
    b2 = _codebook_sqnorms(codebooks)

    idxs = [_round0(r0, codebooks[0], b2[0]).reshape(n)]
    loss_parts = []
    r = r0
    for q in range(1, _NQ):
        quant = jnp.take(codebooks[q - 1], idxs[-1], axis=0)
        idx, r, lp = _round_upd(r, quant, codebooks[q], b2[q])
        idxs.append(idx.reshape(n))
        loss_parts.append(lp)
    quant = jnp.take(codebooks[_NQ - 1], idxs[-1], axis=0)
    rfin, lp = _final_upd(r, quant)
    loss_parts.append(lp)

    qout = (r0 - rfin).reshape(shp)
    indices = jnp.stack([i.reshape(shp[:-1]) for i in idxs], axis=-1)
    denom = float(n * _DIM)
    losses = jnp.stack([jnp.sum(p) / denom for p in loss_parts], axis=-1)
    return qout, indices, losses
